# baseline (speedup 1.0000x reference)
import functools

import jax
import jax.numpy as jnp
from jax import lax
from jax.experimental import pallas as pl
from jax.experimental.pallas import tpu as pltpu
from jax.experimental.pallas import tpu_sc as plsc

_EMBED = 64
_CHUNK = 128


def _make_kernel(batch):
    info = plsc.get_sparse_core_info()
    nc, ns, nl = info.num_cores, info.num_subcores, info.num_lanes
    nw = nc * ns
    bpw = batch // nw
    nchunk = bpw // _CHUNK
    ngroups = bpw // nl

    mesh = plsc.VectorSubcoreMesh(core_axis_name="c", subcore_axis_name="s")

    @functools.partial(
        pl.kernel,
        mesh=mesh,
        out_type=jax.ShapeDtypeStruct((batch,), jnp.float32),
        compiler_params=pltpu.CompilerParams(
            needs_layout_passes=False, use_tc_tiling_on_sc=False),
        scratch_types=[
            pltpu.VMEM((nchunk, _CHUNK), jnp.int32),
            pltpu.VMEM((nchunk, _CHUNK), jnp.int32),
            pltpu.VMEM((nchunk, _CHUNK), jnp.int32),
            pltpu.VMEM((bpw, _EMBED), jnp.float32),
            pltpu.VMEM((bpw, _EMBED), jnp.float32),
            pltpu.VMEM((bpw, _EMBED), jnp.float32),
            pltpu.VMEM((bpw,), jnp.float32),
            pltpu.SemaphoreType.DMA,
        ],
    )
    def scorer(h_hbm, r_hbm, t_hbm, ent_hbm, rel_hbm, out_hbm,
               hidx_v, ridx_v, tidx_v, hrow_v, rrow_v, trow_v, score_v, sem):
        wid = lax.axis_index("s") * nc + lax.axis_index("c")
        base = wid * bpw

        idx_copies = []
        for j in range(nchunk):
            sl = pl.ds(base + j * _CHUNK, _CHUNK)
            idx_copies.append(pltpu.async_copy(h_hbm.at[sl], hidx_v.at[j], sem))
            idx_copies.append(pltpu.async_copy(r_hbm.at[sl], ridx_v.at[j], sem))
            idx_copies.append(pltpu.async_copy(t_hbm.at[sl], tidx_v.at[j], sem))
        for c in idx_copies:
            c.wait()

        row_copies = []
        for j in range(nchunk):
            sl = pl.ds(j * _CHUNK, _CHUNK)
            row_copies.append(
                pltpu.async_copy(ent_hbm.at[hidx_v.at[j]], hrow_v.at[sl], sem))
            row_copies.append(
                pltpu.async_copy(rel_hbm.at[ridx_v.at[j]], rrow_v.at[sl], sem))
            row_copies.append(
                pltpu.async_copy(ent_hbm.at[tidx_v.at[j]], trow_v.at[sl], sem))
        for c in row_copies:
            c.wait()

        lane = lax.iota(jnp.int32, nl)

        def group_body(g, _):
            row0 = pl.multiple_of(g * nl, nl)
            score = jnp.zeros((nl,), jnp.float32)
            for j in range(nl):
                i = row0 + j
                s = jnp.zeros((nl,), jnp.float32)
                for k in range(_EMBED // nl):
                    sl = pl.ds(k * nl, nl)
                    hv = hrow_v[i, sl]
                    rv = rrow_v[i, sl]
                    tv = trow_v[i, sl]
                    delta = hv + rv - tv
                    s = s + delta * delta
                score = jnp.where(lane == j, jnp.sum(s), score)
            score_v[pl.ds(row0, nl)] = score
            return ()

        lax.fori_loop(0, ngroups, group_body, (), unroll=False)

        pltpu.sync_copy(score_v, out_hbm.at[pl.ds(base, bpw)])

    return scorer


def kernel(h, r, t, entity_embeddings, relation_embeddings):
    scorer = _make_kernel(h.shape[0])
    return scorer(h, r, t, entity_embeddings, relation_embeddings)


# speedup vs baseline: 1.4203x; 1.4203x over previous
import functools

import jax
import jax.numpy as jnp
from jax import lax
from jax.experimental import pallas as pl
from jax.experimental.pallas import tpu as pltpu
from jax.experimental.pallas import tpu_sc as plsc

_EMBED = 64
_SUBROWS = 8
_CHUNK = 32


def _make_kernel(batch):
    info = plsc.get_sparse_core_info()
    nc, ns, nl = info.num_cores, info.num_subcores, info.num_lanes
    nw = nc * ns
    bpw = batch // nw
    nchunk = bpw // _CHUNK

    mesh = plsc.VectorSubcoreMesh(core_axis_name="c", subcore_axis_name="s")

    @functools.partial(
        pl.kernel,
        mesh=mesh,
        out_type=jax.ShapeDtypeStruct((batch,), jnp.float32),
        compiler_params=pltpu.CompilerParams(
            needs_layout_passes=False, use_tc_tiling_on_sc=True),
        scratch_types=[
            pltpu.VMEM((bpw,), jnp.int32),
            pltpu.VMEM((bpw,), jnp.int32),
            pltpu.VMEM((bpw,), jnp.int32),
            pltpu.VMEM((_CHUNK, _SUBROWS, _EMBED), jnp.float32),
            pltpu.VMEM((_CHUNK, _SUBROWS, _EMBED), jnp.float32),
            pltpu.VMEM((_CHUNK, _SUBROWS, _EMBED), jnp.float32),
            pltpu.VMEM((bpw,), jnp.float32),
            pltpu.SemaphoreType.DMA,
        ],
    )
    def scorer(h_hbm, r_hbm, t_hbm, ent_hbm, rel_hbm, out_hbm,
               hidx_v, ridx_v, tidx_v, hblk_v, rblk_v, tblk_v, score_v, sem):
        wid = lax.axis_index("s") * nc + lax.axis_index("c")
        base = wid * bpw
        osl = pl.ds(base, bpw)
        ci = pltpu.async_copy(h_hbm.at[osl], hidx_v, sem)
        cr = pltpu.async_copy(r_hbm.at[osl], ridx_v, sem)
        ct = pltpu.async_copy(t_hbm.at[osl], tidx_v, sem)
        ci.wait()
        cr.wait()
        ct.wait()

        lane = lax.iota(jnp.int32, nl)

        def chunk_body(c, _):
            e0 = pl.multiple_of(c * _CHUNK, _CHUNK)
            idx16 = []
            for g in range(_CHUNK // nl):
                gsl = pl.ds(e0 + g * nl, nl)
                idx16.append((hidx_v[gsl], ridx_v[gsl], tidx_v[gsl]))

            for g, (ih16, ir16, it16) in enumerate(idx16):
                for j in range(nl):
                    el = g * nl + j
                    bh = pl.multiple_of((ih16[j] // _SUBROWS) * _SUBROWS,
                                        _SUBROWS)
                    br = pl.multiple_of((ir16[j] // _SUBROWS) * _SUBROWS,
                                        _SUBROWS)
                    bt = pl.multiple_of((it16[j] // _SUBROWS) * _SUBROWS,
                                        _SUBROWS)
                    pltpu.async_copy(
                        ent_hbm.at[pl.ds(bh, _SUBROWS)], hblk_v.at[el], sem)
                    pltpu.async_copy(
                        rel_hbm.at[pl.ds(br, _SUBROWS)], rblk_v.at[el], sem)
                    pltpu.async_copy(
                        ent_hbm.at[pl.ds(bt, _SUBROWS)], tblk_v.at[el], sem)

            for el in range(_CHUNK):
                pltpu.make_async_copy(
                    ent_hbm.at[pl.ds(0, _SUBROWS)], hblk_v.at[el], sem).wait()
                pltpu.make_async_copy(
                    rel_hbm.at[pl.ds(0, _SUBROWS)], rblk_v.at[el], sem).wait()
                pltpu.make_async_copy(
                    ent_hbm.at[pl.ds(0, _SUBROWS)], tblk_v.at[el], sem).wait()

            for g, (ih16, ir16, it16) in enumerate(idx16):
                ihm = lax.rem(ih16, _SUBROWS)
                irm = lax.rem(ir16, _SUBROWS)
                itm = lax.rem(it16, _SUBROWS)
                score = jnp.zeros((nl,), jnp.float32)
                for j in range(nl):
                    el = g * nl + j
                    s = jnp.zeros((nl,), jnp.float32)
                    for k in range(_EMBED // nl):
                        dsl = pl.ds(k * nl, nl)
                        hv = hblk_v[el, ihm[j], dsl]
                        rv = rblk_v[el, irm[j], dsl]
                        tv = tblk_v[el, itm[j], dsl]
                        delta = hv + rv - tv
                        s = s + delta * delta
                    score = jnp.where(lane == j, jnp.sum(s), score)
                score_v[pl.ds(e0 + g * nl, nl)] = score
            return ()

        lax.fori_loop(0, nchunk, chunk_body, (), unroll=False)

        pltpu.sync_copy(score_v, out_hbm.at[osl])

    return scorer


def kernel(h, r, t, entity_embeddings, relation_embeddings):
    scorer = _make_kernel(h.shape[0])
    return scorer(h, r, t, entity_embeddings, relation_embeddings)
